# Initial kernel scaffold; baseline (speedup 1.0000x reference)
import jax
import jax.numpy as jnp
from jax.experimental import pallas as pl


def kernel(indices, emb_table, W_out, b_out):
    raise NotImplementedError("write your pallas kernel here")


# speedup vs baseline: 16.0988x; 16.0988x over previous
import functools

import jax
import jax.numpy as jnp
from jax import lax
from jax.experimental import pallas as pl
from jax.experimental.pallas import tpu as pltpu
from jax.experimental.pallas import tpu_sc as plsc

_VOCAB = 100000
_EMB = 64
_BATCH = 4096
_SEQ = 200
_NOUT = 10

_NC = 2
_NS = 16
_NW = _NC * _NS
_ROWS_PER_W = _BATCH // _NW
_CHUNK = 104
_SEQ_PAD = 2 * _CHUNK
_CHUNKS_PER_W = 2 * _ROWS_PER_W
_NBUF = 4


def _sc_pool(idx_rs, emb_table):
    mesh = plsc.VectorSubcoreMesh(core_axis_name="c", subcore_axis_name="s")

    @functools.partial(
        pl.kernel,
        mesh=mesh,
        out_type=jax.ShapeDtypeStruct((_BATCH, _EMB), jnp.float32),
        scratch_types=[
            pltpu.VMEM((_CHUNKS_PER_W, _CHUNK), jnp.int32),
            pltpu.VMEM((_CHUNK, _EMB), jnp.float32),
            pltpu.VMEM((_CHUNK, _EMB), jnp.float32),
            pltpu.VMEM((_CHUNK, _EMB), jnp.float32),
            pltpu.VMEM((_CHUNK, _EMB), jnp.float32),
            pltpu.VMEM((_ROWS_PER_W, _EMB), jnp.float32),
            pltpu.SemaphoreType.DMA,
            pltpu.SemaphoreType.DMA,
            pltpu.SemaphoreType.DMA,
            pltpu.SemaphoreType.DMA,
        ],
        compiler_params=pltpu.CompilerParams(use_tc_tiling_on_sc=False),
    )
    def pool_kernel(idx_hbm, table_hbm, out_hbm,
                    idx_v, buf0, buf1, buf2, buf3, outs_v,
                    sem0, sem1, sem2, sem3):
        bufs = (buf0, buf1, buf2, buf3)
        sems = (sem0, sem1, sem2, sem3)
        wid = lax.axis_index("s") * _NC + lax.axis_index("c")

        pltpu.sync_copy(idx_hbm.at[wid], idx_v)

        def start(k, b):
            pltpu.make_async_copy(
                table_hbm.at[idx_v.at[k]], bufs[b], sems[b]).start()

        def wait(k, b):
            pltpu.make_async_copy(
                table_hbm.at[idx_v.at[k]], bufs[b], sems[b]).wait()

        for b in range(_NBUF):
            start(b, b)

        neg = jnp.full((16,), -jnp.inf, dtype=jnp.float32)

        def reduce_chunk(buf, accs):
            def body(t, accs):
                a0, a1, a2, a3 = accs
                for u in range(8):
                    r = t * 8 + u
                    a0 = jnp.maximum(a0, buf[r, pl.ds(0, 16)])
                    a1 = jnp.maximum(a1, buf[r, pl.ds(16, 16)])
                    a2 = jnp.maximum(a2, buf[r, pl.ds(32, 16)])
                    a3 = jnp.maximum(a3, buf[r, pl.ds(48, 16)])
                return a0, a1, a2, a3
            return lax.fori_loop(0, _CHUNK // 8, body, accs)

        def store_row(row, accs):
            a0, a1, a2, a3 = accs
            outs_v[row, pl.ds(0, 16)] = a0
            outs_v[row, pl.ds(16, 16)] = a1
            outs_v[row, pl.ds(32, 16)] = a2
            outs_v[row, pl.ds(48, 16)] = a3

        def group(g, carry):
            for b in range(_NBUF):
                k = _NBUF * g + b
                wait(k, b)
                if b % 2 == 0:
                    accs = reduce_chunk(bufs[b], (neg, neg, neg, neg))
                else:
                    accs = reduce_chunk(bufs[b], accs)
                    store_row(2 * g + b // 2, accs)
                nk = k + _NBUF

                @pl.when(nk < _CHUNKS_PER_W)
                def _start_next():
                    start(nk, b)
            return carry

        lax.fori_loop(0, _CHUNKS_PER_W // _NBUF, group, 0)

        pltpu.sync_copy(outs_v, out_hbm.at[pl.ds(wid * _ROWS_PER_W,
                                                 _ROWS_PER_W)])

    return pool_kernel


def _tc_head(pooled, W_out, b_out):
    def body(x_ref, w_ref, b_ref, o_ref):
        logits = jnp.dot(x_ref[...], w_ref[...],
                         preferred_element_type=jnp.float32) + b_ref[...]
        m = jnp.max(logits, axis=-1, keepdims=True)
        e = jnp.exp(logits - m)
        o_ref[...] = e / jnp.sum(e, axis=-1, keepdims=True)

    return pl.pallas_call(
        body,
        out_shape=jax.ShapeDtypeStruct((_BATCH, _NOUT), jnp.float32),
    )(pooled, W_out, b_out.reshape(1, _NOUT))


def kernel(indices, emb_table, W_out, b_out):
    idx_pad = jnp.concatenate([indices, indices[:, : _SEQ_PAD - _SEQ]], axis=1)
    idx_rs = idx_pad.reshape(_NW, _CHUNKS_PER_W, _CHUNK)
    pooled = _sc_pool(idx_rs, emb_table)(idx_rs, emb_table)
    return _tc_head(pooled, W_out, b_out)
